# Initial kernel scaffold; baseline (speedup 1.0000x reference)
import jax
import jax.numpy as jnp
from jax.experimental import pallas as pl


def kernel(x, table):
    raise NotImplementedError("write your pallas kernel here")


# speedup vs baseline: 1.1095x; 1.1095x over previous
import functools

import jax
import jax.numpy as jnp
from jax import lax
from jax.experimental import pallas as pl
from jax.experimental.pallas import tpu as pltpu
from jax.experimental.pallas import tpu_sc as plsc

_VOCAB = 1000000
_DIM = 32
_ROWS = 16384
_COLS = 50
_B = _ROWS * _COLS

_NC = 2
_NS = 16
_NW = _NC * _NS
_B_PER_W = _B // _NW
_CHUNK = 1280
_NCHUNKS = _B_PER_W // _CHUNK

_mesh = plsc.VectorSubcoreMesh(core_axis_name="c", subcore_axis_name="s")


@functools.partial(
    pl.kernel,
    out_type=jax.ShapeDtypeStruct((_B, _DIM), jnp.float32),
    mesh=_mesh,
    compiler_params=pltpu.CompilerParams(use_tc_tiling_on_sc=False),
    scratch_types=[
        pltpu.VMEM((_B_PER_W,), jnp.int32),
        pltpu.VMEM((_CHUNK, _DIM), jnp.float32),
        pltpu.VMEM((_CHUNK, _DIM), jnp.float32),
        pltpu.SemaphoreType.DMA,
    ],
)
def _gather_kernel(idx_hbm, table_hbm, out_hbm, idx_v, rows0, rows1, gsem):
    wid = lax.axis_index("s") * _NC + lax.axis_index("c")
    base = wid * _B_PER_W
    pltpu.sync_copy(idx_hbm.at[pl.ds(base, _B_PER_W)], idx_v)

    bufs = (rows0, rows1)
    handles = [None, None]
    handles[0] = pltpu.async_copy(
        table_hbm.at[idx_v.at[pl.ds(0, _CHUNK)]], bufs[0], gsem)
    for c in range(_NCHUNKS):
        b = c % 2
        handles[b].wait()
        if c + 1 < _NCHUNKS:
            handles[1 - b] = pltpu.async_copy(
                table_hbm.at[idx_v.at[pl.ds((c + 1) * _CHUNK, _CHUNK)]],
                bufs[1 - b], gsem)
        pltpu.sync_copy(bufs[b], out_hbm.at[pl.ds(base + c * _CHUNK, _CHUNK)])


def kernel(x, table):
    flat = _gather_kernel(x.reshape(_B), table)
    return flat.reshape(_ROWS, _COLS, _DIM)


# speedup vs baseline: 1.7568x; 1.5834x over previous
import functools

import jax
import jax.numpy as jnp
from jax import lax
from jax.experimental import pallas as pl
from jax.experimental.pallas import tpu as pltpu
from jax.experimental.pallas import tpu_sc as plsc

_VOCAB = 1000000
_DIM = 32
_ROWS = 16384
_COLS = 50
_PADL = 128
_PADC = 56

_NC = 2
_NS = 16
_NW = _NC * _NS
_RPW = _ROWS // _NW
_G = 16
_GL = 56
_NCHUNKS = _RPW // _G

_mesh = plsc.VectorSubcoreMesh(core_axis_name="c", subcore_axis_name="s")


@functools.partial(
    pl.kernel,
    out_type=jax.ShapeDtypeStruct((_ROWS, _COLS, _DIM), jnp.float32),
    mesh=_mesh,
    compiler_params=pltpu.CompilerParams(use_tc_tiling_on_sc=False),
    scratch_types=[
        pltpu.VMEM((_RPW, _PADL), jnp.int32),
        pltpu.VMEM((_G * _GL, _DIM), jnp.float32),
        pltpu.VMEM((_G * _GL, _DIM), jnp.float32),
        pltpu.SemaphoreType.DMA,
        pltpu.SemaphoreType.DMA,
        pltpu.SemaphoreType.DMA,
    ],
)
def _gather_kernel(x_hbm, table_hbm, out_hbm, xrows_v, rows0, rows1,
                   gsem, ssem0, ssem1):
    wid = lax.axis_index("s") * _NC + lax.axis_index("c")
    base = wid * _RPW
    pltpu.sync_copy(x_hbm.at[pl.ds(base, _RPW)], xrows_v)

    bufs = (rows0, rows1)
    ssems = (ssem0, ssem1)
    ghandles = [[], []]
    shandles = [[], []]

    def fire(c):
        b = c % 2
        for j in range(_G):
            r = c * _G + j
            ghandles[b].append(pltpu.async_copy(
                table_hbm.at[xrows_v.at[r, pl.ds(0, _GL)]],
                bufs[b].at[pl.ds(j * _GL, _GL)], gsem))

    fire(0)
    for c in range(_NCHUNKS):
        b = c % 2
        for h in ghandles[b]:
            h.wait()
        ghandles[b] = []
        for h in shandles[1 - b]:
            h.wait()
        shandles[1 - b] = []
        if c + 1 < _NCHUNKS:
            fire(c + 1)
        row0 = base + c * _G
        for j in range(_G):
            shandles[b].append(pltpu.async_copy(
                bufs[b].at[pl.ds(j * _GL, _COLS)],
                out_hbm.at[row0 + j], ssems[b]))
    for h in shandles[0] + shandles[1]:
        h.wait()


def kernel(x, table):
    x128 = jnp.pad(x, ((0, 0), (0, _PADL - _COLS)), mode="edge")
    return _gather_kernel(x128, table)


# speedup vs baseline: 2.4535x; 1.3966x over previous
import functools

import jax
import jax.numpy as jnp
from jax import lax
from jax.experimental import pallas as pl
from jax.experimental.pallas import tpu as pltpu
from jax.experimental.pallas import tpu_sc as plsc

_VOCAB = 1000000
_DIM = 32
_ROWS = 16384
_COLS = 50
_PADL = 128
_PADC = 56

_NC = 2
_NS = 16
_NW = _NC * _NS
_RPW = _ROWS // _NW
_G = 16
_GL = 56
_NCHUNKS = _RPW // _G

_mesh = plsc.VectorSubcoreMesh(core_axis_name="c", subcore_axis_name="s")


@functools.partial(
    pl.kernel,
    out_type=jax.ShapeDtypeStruct((_ROWS, _PADC, _PADL), jnp.float32),
    mesh=_mesh,
    compiler_params=pltpu.CompilerParams(use_tc_tiling_on_sc=False),
    scratch_types=[
        pltpu.VMEM((_RPW, _PADL), jnp.int32),
        pltpu.VMEM((_G * _GL, _DIM), jnp.float32),
        pltpu.VMEM((_G * _GL, _DIM), jnp.float32),
        pltpu.SemaphoreType.DMA,
        pltpu.SemaphoreType.DMA,
        pltpu.SemaphoreType.DMA,
    ],
)
def _gather_kernel(x_hbm, table_hbm, out_hbm, xrows_v, rows0, rows1,
                   gsem, ssem0, ssem1):
    wid = lax.axis_index("s") * _NC + lax.axis_index("c")
    base = wid * _RPW
    pltpu.sync_copy(x_hbm.at[pl.ds(base, _RPW)], xrows_v)

    bufs = (rows0, rows1)
    ssems = (ssem0, ssem1)
    ghandles = [[], []]
    shandles = [[], []]

    def fire(c):
        b = c % 2
        for j in range(_G):
            r = c * _G + j
            ghandles[b].append(pltpu.async_copy(
                table_hbm.at[xrows_v.at[r, pl.ds(0, _GL)]],
                bufs[b].at[pl.ds(j * _GL, _GL)], gsem))

    fire(0)
    for c in range(_NCHUNKS):
        b = c % 2
        for h in ghandles[b]:
            h.wait()
        ghandles[b] = []
        for h in shandles[1 - b]:
            h.wait()
        shandles[1 - b] = []
        if c + 1 < _NCHUNKS:
            fire(c + 1)
        row0 = base + c * _G
        for j in range(_G):
            shandles[b].append(pltpu.async_copy(
                bufs[b].at[pl.ds(j * _GL, _COLS)],
                out_hbm.at[row0 + j, pl.ds(0, _COLS), pl.ds(0, _DIM)],
                ssems[b]))
    for h in shandles[0] + shandles[1]:
        h.wait()


def kernel(x, table):
    x128 = jnp.pad(x, ((0, 0), (0, _PADL - _COLS)), mode="edge")
    mid = _gather_kernel(x128, table)
    return mid[:, :_COLS, :_DIM]
